# baseline (speedup 1.0000x reference)
import jax
import jax.numpy as jnp
from jax import lax
from jax.experimental import pallas as pl
from jax.experimental.pallas import tpu as pltpu
from jax.experimental.pallas import tpu_sc as plsc

_L = 16


def _sc_body(p_hbm, out_hbm, p_v, o_v):
    cid = lax.axis_index("c")
    sid = lax.axis_index("s")

    @pl.when(jnp.logical_and(cid == 0, sid == 0))
    def _():
        pltpu.sync_copy(p_hbm, p_v)
        p = p_v[...]
        lane = lax.iota(jnp.int32, _L)
        c_s = jnp.sum(jnp.where(lane == 11, p, 0.0)).astype(jnp.int32)
        b_s = jnp.sum(jnp.where(lane == 10, p, 0.0))
        w_c = plsc.load_gather(p_v, [jnp.full((_L,), c_s)])
        o_v[...] = w_c + b_s
        pltpu.sync_copy(o_v, out_hbm)


def kernel(c, W, b):
    c_f = jnp.asarray(c, jnp.float32).reshape(1)
    packed = jnp.concatenate(
        [W.reshape(-1), b, c_f, jnp.zeros((_L - 12,), jnp.float32)]
    )
    mesh = plsc.VectorSubcoreMesh(
        core_axis_name="c", subcore_axis_name="s", num_cores=1, num_subcores=1
    )
    out16 = pl.kernel(
        _sc_body,
        out_type=jax.ShapeDtypeStruct((_L,), jnp.float32),
        mesh=mesh,
        compiler_params=pltpu.CompilerParams(
            needs_layout_passes=False, skip_device_barrier=True
        ),
        scratch_types=[
            pltpu.VMEM((_L,), jnp.float32),
            pltpu.VMEM((_L,), jnp.float32),
        ],
    )(packed)
    return out16[:1]
